# baseline (speedup 1.0000x reference)
import functools

import jax
import jax.numpy as jnp
from jax import lax
from jax.experimental import pallas as pl
from jax.experimental.pallas import tpu as pltpu
from jax.experimental.pallas import tpu_sc as plsc

E = 320000
N = 10000
D = 128
L = 16
NC = 2
NS = 16
NW = NC * NS
EPW = E // NW
C = 80
NCHUNK = EPW // C
GPC = C // L
BM = 400


def _gram_block(zb_ref, zfull_ref, g_ref):
    g_ref[...] = lax.dot_general(
        zb_ref[...], zfull_ref[...],
        dimension_numbers=(((1,), (1,)), ((), ())),
        preferred_element_type=jnp.float32)


_GRAM = pl.pallas_call(
    _gram_block,
    grid=(N // BM,),
    in_specs=[
        pl.BlockSpec((BM, D), lambda i: (i, 0)),
        pl.BlockSpec((N, D), lambda i: (0, 0)),
    ],
    out_specs=pl.BlockSpec((BM, N), lambda i: (i, 0)),
    out_shape=jax.ShapeDtypeStruct((N, N), jnp.float32),
)


def _make_sc_gather():
    mesh = plsc.VectorSubcoreMesh(core_axis_name="c", subcore_axis_name="s")

    @functools.partial(
        pl.kernel,
        mesh=mesh,
        out_type=jax.ShapeDtypeStruct((E,), jnp.float32),
        scratch_types=[
            pltpu.VMEM((NCHUNK, C), jnp.int32),
            pltpu.VMEM((C,), jnp.float32),
            pltpu.VMEM((C,), jnp.float32),
            pltpu.VMEM((EPW,), jnp.float32),
            pltpu.SemaphoreType.DMA,
            pltpu.SemaphoreType.DMA,
        ],
    )
    def sc_call(gflat_hbm, fidx_hbm, out_hbm,
                fidx_v, vals_a, vals_b, outc_v, sem_a, sem_b):
        wid = lax.axis_index("s") * NC + lax.axis_index("c")
        pltpu.sync_copy(fidx_hbm.at[wid], fidx_v)

        def sigmoid_chunk(vals, i):
            def group_body(g, carry):
                s = vals[pl.ds(g * L, L)]
                outc_v[pl.ds(i * C + g * L, L)] = 1.0 / (1.0 + jnp.exp(-s))
                return carry
            lax.fori_loop(0, GPC, group_body, 0)

        def gather_start(i, vals, sem):
            pltpu.make_async_copy(gflat_hbm.at[fidx_v.at[i]], vals, sem).start()

        def gather_wait(i, vals, sem):
            pltpu.make_async_copy(gflat_hbm.at[fidx_v.at[i]], vals, sem).wait()

        gather_start(0, vals_a, sem_a)

        def pair_body(j, carry):
            i0 = 2 * j
            i1 = i0 + 1
            gather_start(i1, vals_b, sem_b)
            gather_wait(i0, vals_a, sem_a)
            sigmoid_chunk(vals_a, i0)
            gather_start(i0 + 2, vals_a, sem_a)
            gather_wait(i1, vals_b, sem_b)
            sigmoid_chunk(vals_b, i1)
            return carry

        lax.fori_loop(0, (NCHUNK - 1) // 2, pair_body, 0)
        last = NCHUNK - 1
        gather_wait(last, vals_a, sem_a)
        sigmoid_chunk(vals_a, last)

        pltpu.sync_copy(outc_v, out_hbm.at[pl.ds(wid * EPW, EPW)])

    return sc_call


_SC_GATHER = _make_sc_gather()


def kernel(z, edge_index):
    ei = edge_index.astype(jnp.int32)
    fidx = (ei[0] * N + ei[1]).reshape(NW, NCHUNK, C)
    g = _GRAM(z, z)
    return _SC_GATHER(g.reshape(N * N), fidx)


# speedup vs baseline: 1.2204x; 1.2204x over previous
import functools

import jax
import jax.numpy as jnp
from jax import lax
from jax.experimental import pallas as pl
from jax.experimental.pallas import tpu as pltpu
from jax.experimental.pallas import tpu_sc as plsc

E = 320000
D = 128
L = 16
NC = 2
NS = 16
NW = NC * NS
EPW = E // NW
C = 80
NCHUNK = EPW // C
GPC = C // L
KD = D // L


def _compute_chunk(rows_v, cols_v, outc_v, i):
    lane = lax.broadcasted_iota(jnp.int32, (L,), 0)

    def group_body(g, carry):
        sel = None
        for e in range(L):
            eoff = g * L + e
            acc = rows_v[eoff, pl.ds(0, L)] * cols_v[eoff, pl.ds(0, L)]
            for k in range(1, KD):
                acc = acc + (rows_v[eoff, pl.ds(k * L, L)]
                             * cols_v[eoff, pl.ds(k * L, L)])
            for shift in (8, 4, 2, 1):
                acc = acc + acc.at[lane ^ shift].get(mode="promise_in_bounds")
            sel = acc if e == 0 else jnp.where(lane == e, acc, sel)
        sig = 1.0 / (1.0 + jnp.exp(-sel))
        outc_v[pl.ds(i * C + g * L, L)] = sig
        return carry

    lax.fori_loop(0, GPC, group_body, 0, unroll=True)


def _make_sc_call():
    mesh = plsc.VectorSubcoreMesh(core_axis_name="c", subcore_axis_name="s")

    @functools.partial(
        pl.kernel,
        mesh=mesh,
        out_type=jax.ShapeDtypeStruct((E,), jnp.float32),
        scratch_types=[
            pltpu.VMEM((NCHUNK, C), jnp.int32),
            pltpu.VMEM((NCHUNK, C), jnp.int32),
            pltpu.VMEM((C, D), jnp.float32),
            pltpu.VMEM((C, D), jnp.float32),
            pltpu.VMEM((C, D), jnp.float32),
            pltpu.VMEM((C, D), jnp.float32),
            pltpu.VMEM((EPW,), jnp.float32),
            pltpu.SemaphoreType.DMA,
            pltpu.SemaphoreType.DMA,
            pltpu.SemaphoreType.DMA,
            pltpu.SemaphoreType.DMA,
        ],
    )
    def sc_call(z_hbm, row_hbm, col_hbm, out_hbm,
                ridx_v, cidx_v, rows_a, cols_a, rows_b, cols_b,
                outc_v, sem_ra, sem_ca, sem_rb, sem_cb):
        wid = lax.axis_index("s") * NC + lax.axis_index("c")
        pltpu.sync_copy(row_hbm.at[wid], ridx_v)
        pltpu.sync_copy(col_hbm.at[wid], cidx_v)

        def gather_start(i, rows, cols, sr, sc2):
            pltpu.make_async_copy(z_hbm.at[ridx_v.at[i]], rows, sr).start()
            pltpu.make_async_copy(z_hbm.at[cidx_v.at[i]], cols, sc2).start()

        def gather_wait(i, rows, cols, sr, sc2):
            pltpu.make_async_copy(z_hbm.at[ridx_v.at[i]], rows, sr).wait()
            pltpu.make_async_copy(z_hbm.at[cidx_v.at[i]], cols, sc2).wait()

        gather_start(0, rows_a, cols_a, sem_ra, sem_ca)

        def pair_body(j, carry):
            i0 = 2 * j
            i1 = i0 + 1
            gather_start(i1, rows_b, cols_b, sem_rb, sem_cb)
            gather_wait(i0, rows_a, cols_a, sem_ra, sem_ca)
            _compute_chunk(rows_a, cols_a, outc_v, i0)
            gather_start(i0 + 2, rows_a, cols_a, sem_ra, sem_ca)
            gather_wait(i1, rows_b, cols_b, sem_rb, sem_cb)
            _compute_chunk(rows_b, cols_b, outc_v, i1)
            return carry

        lax.fori_loop(0, (NCHUNK - 1) // 2, pair_body, 0)
        last = NCHUNK - 1
        gather_wait(last, rows_a, cols_a, sem_ra, sem_ca)
        _compute_chunk(rows_a, cols_a, outc_v, last)

        pltpu.sync_copy(outc_v, out_hbm.at[pl.ds(wid * EPW, EPW)])

    return sc_call


_SC_CALL = _make_sc_call()


def kernel(z, edge_index):
    ei = edge_index.astype(jnp.int32)
    row2 = ei[0].reshape(NW, NCHUNK, C)
    col2 = ei[1].reshape(NW, NCHUNK, C)
    return _SC_CALL(z, row2, col2)


# speedup vs baseline: 2.7368x; 2.2426x over previous
import functools

import jax
import jax.numpy as jnp
from jax import lax
from jax.experimental import pallas as pl
from jax.experimental.pallas import tpu as pltpu
from jax.experimental.pallas import tpu_sc as plsc

E = 320000
D = 128
L = 16
NC = 2
NS = 16
NW = NC * NS
EPW = E // NW
C = 80
NCHUNK = EPW // C
GPC = C // L
KD = D // L


def _compute_chunk(rows_v, cols_v, outc_v, i):
    lane = lax.broadcasted_iota(jnp.int32, (L,), 0)

    def group_body(g, sel_prev):
        sel = None
        for e in range(L // 2):
            eoff = g * (L // 2) + e
            acc = rows_v[eoff, pl.ds(0, L)] * cols_v[eoff, pl.ds(0, L)]
            for k in range(1, KD):
                acc = acc + (rows_v[eoff, pl.ds(k * L, L)]
                             * cols_v[eoff, pl.ds(k * L, L)])
            for shift in (8, 4, 2, 1):
                acc = acc + acc.at[lane ^ shift].get(mode="promise_in_bounds")
            sel = acc if e == 0 else jnp.where(lane == e, acc, sel)

        @pl.when(g % 2 == 1)
        def _store():
            hi = sel.at[lane ^ (L // 2)].get(mode="promise_in_bounds")
            both = jnp.where(lane < (L // 2), sel_prev, hi)
            outc_v[pl.ds(i * C + (g // 2) * L, L)] = 1.0 / (1.0 + jnp.exp(-both))

        return sel

    lax.fori_loop(0, 2 * GPC, group_body, jnp.zeros((L,), jnp.float32))


def _make_sc_call():
    mesh = plsc.VectorSubcoreMesh(core_axis_name="c", subcore_axis_name="s")

    @functools.partial(
        pl.kernel,
        mesh=mesh,
        out_type=jax.ShapeDtypeStruct((E,), jnp.float32),
        scratch_types=[
            pltpu.VMEM((NCHUNK, C), jnp.int32),
            pltpu.VMEM((NCHUNK, C), jnp.int32),
            pltpu.VMEM((C, D), jnp.float32),
            pltpu.VMEM((C, D), jnp.float32),
            pltpu.VMEM((C, D), jnp.float32),
            pltpu.VMEM((C, D), jnp.float32),
            pltpu.VMEM((EPW + 8,), jnp.float32),
            pltpu.SemaphoreType.DMA,
            pltpu.SemaphoreType.DMA,
            pltpu.SemaphoreType.DMA,
            pltpu.SemaphoreType.DMA,
        ],
    )
    def sc_call(z_hbm, row_hbm, col_hbm, out_hbm,
                ridx_v, cidx_v, rows_a, cols_a, rows_b, cols_b,
                outc_v, sem_ra, sem_ca, sem_rb, sem_cb):
        wid = lax.axis_index("s") * NC + lax.axis_index("c")
        pltpu.sync_copy(row_hbm.at[wid], ridx_v)
        pltpu.sync_copy(col_hbm.at[wid], cidx_v)

        def gather_start(i, rows, cols, sr, sc2):
            pltpu.make_async_copy(z_hbm.at[ridx_v.at[i]], rows, sr).start()
            pltpu.make_async_copy(z_hbm.at[cidx_v.at[i]], cols, sc2).start()

        def gather_wait(i, rows, cols, sr, sc2):
            pltpu.make_async_copy(z_hbm.at[ridx_v.at[i]], rows, sr).wait()
            pltpu.make_async_copy(z_hbm.at[cidx_v.at[i]], cols, sc2).wait()

        gather_start(0, rows_a, cols_a, sem_ra, sem_ca)

        def pair_body(j, carry):
            i0 = 2 * j
            i1 = i0 + 1
            gather_start(i1, rows_b, cols_b, sem_rb, sem_cb)
            gather_wait(i0, rows_a, cols_a, sem_ra, sem_ca)
            _compute_chunk(rows_a, cols_a, outc_v, i0)
            gather_start(i0 + 2, rows_a, cols_a, sem_ra, sem_ca)
            gather_wait(i1, rows_b, cols_b, sem_rb, sem_cb)
            _compute_chunk(rows_b, cols_b, outc_v, i1)
            return carry

        lax.fori_loop(0, (NCHUNK - 1) // 2, pair_body, 0)
        last = NCHUNK - 1
        gather_wait(last, rows_a, cols_a, sem_ra, sem_ca)
        _compute_chunk(rows_a, cols_a, outc_v, last)

        pltpu.sync_copy(outc_v.at[pl.ds(0, EPW)],
                        out_hbm.at[pl.ds(wid * EPW, EPW)])

    return sc_call


_SC_CALL = _make_sc_call()


def kernel(z, edge_index):
    ei = edge_index.astype(jnp.int32)
    row2 = ei[0].reshape(NW, NCHUNK, C)
    col2 = ei[1].reshape(NW, NCHUNK, C)
    return _SC_CALL(z, row2, col2)
